# baseline (speedup 1.0000x reference)
import functools

import jax
import jax.numpy as jnp
from jax import lax
from jax.experimental import pallas as pl
from jax.experimental.pallas import tpu as pltpu
from jax.experimental.pallas import tpu_sc as plsc

NC = 2
NS = 16
LANES = 16
NBUF = 3


def _comb_table(segment_embeddings, seq_len):
    n_seg, d = segment_embeddings.shape

    def body(seg_ref, out_ref):
        s_idx = lax.broadcasted_iota(jnp.int32, (seq_len, d), 0)
        d_idx = lax.broadcasted_iota(jnp.int32, (seq_len, d), 1)
        i2 = ((d_idx // 2) * 2).astype(jnp.float32)
        div = jnp.exp(-jnp.log(10000.0) * i2 / d)
        ang = s_idx.astype(jnp.float32) * div
        pe = jnp.where(d_idx % 2 == 0, jnp.sin(ang), jnp.cos(ang))
        for t in range(n_seg):
            out_ref[pl.ds(t * seq_len, seq_len), :] = pe + seg_ref[t:t + 1, :]

    return pl.pallas_call(
        body,
        out_shape=jax.ShapeDtypeStruct((n_seg * seq_len, d), jnp.float32),
    )(segment_embeddings)


def _sc_embed(tok_flat, seg_flat, word_embeddings, comb_flat, seq_len):
    n = tok_flat.shape[0]
    d = word_embeddings.shape[1]
    nw = NC * NS
    rows_per_w = n // nw
    ch = 160
    n_chunks = rows_per_w // ch
    n_rounds, n_tail = divmod(n_chunks, NBUF)
    comb_elems = comb_flat.shape[0]
    assert rows_per_w % ch == 0 and ch % LANES == 0 and n_chunks >= NBUF

    mesh = plsc.VectorSubcoreMesh(
        core_axis_name="c", subcore_axis_name="s",
        num_cores=NC, num_subcores=NS)

    @functools.partial(
        pl.kernel,
        out_type=jax.ShapeDtypeStruct((n, d), jnp.float32),
        mesh=mesh,
        scratch_types=[
            pltpu.VMEM((rows_per_w,), jnp.int32),
            pltpu.VMEM((rows_per_w + LANES,), jnp.int32),
            pltpu.VMEM((comb_elems,), jnp.float32),
            [pltpu.VMEM((ch, d), jnp.float32) for _ in range(NBUF)],
            [pltpu.SemaphoreType.DMA for _ in range(NBUF)],
            [pltpu.SemaphoreType.DMA for _ in range(NBUF)],
        ],
    )
    def k(tok_hbm, seg_hbm, wtab_hbm, comb_hbm, out_hbm,
          tokb, segb, combl, bufs, gsems, wsems):
        wid = lax.axis_index("s") * NC + lax.axis_index("c")
        base = wid * rows_per_w

        pltpu.sync_copy(tok_hbm.at[pl.ds(base, rows_per_w)], tokb)

        def start_gather(b, c):
            pltpu.async_copy(
                wtab_hbm.at[tokb.at[pl.ds(c * ch, ch)]], bufs[b], gsems[b])

        def wait_gather(b):
            pltpu.make_async_copy(
                wtab_hbm.at[pl.ds(0, ch)], bufs[b], gsems[b]).wait()

        def start_write(b, c):
            pltpu.async_copy(
                bufs[b], out_hbm.at[pl.ds(base + c * ch, ch)], wsems[b])

        def wait_write(b):
            pltpu.make_async_copy(
                bufs[b], out_hbm.at[pl.ds(base, ch)], wsems[b]).wait()

        def add_comb(b, c):
            buf = bufs[b]

            @plsc.parallel_loop(0, ch // LANES, unroll=1)
            def _(g):
                gbase = g * LANES
                seg_v = segb[pl.ds(c * ch + gbase, LANES)]
                for r in range(LANES):
                    row = gbase + r
                    flat = c * ch + row
                    off = (seg_v[r] * seq_len + lax.rem(flat, seq_len)) * d
                    for j in range(d // LANES):
                        cv = combl[pl.ds(off + j * LANES, LANES)]
                        plsc.addupdate(buf.at[row, pl.ds(j * LANES, LANES)], cv)

        for b in range(NBUF):
            start_gather(b, b)
        pltpu.sync_copy(comb_hbm, combl)
        pltpu.sync_copy(seg_hbm.at[pl.ds(base, rows_per_w)],
                        segb.at[pl.ds(0, rows_per_w)])

        def process(cc, b):
            prev = (b - 1) % NBUF

            def refill():
                wait_write(prev)
                start_gather(prev, cc + NBUF - 1)

            if isinstance(cc, int):
                if 1 <= cc and cc + NBUF - 1 < n_chunks:
                    refill()
            else:
                pl.when(jnp.logical_and(cc >= 1, cc + NBUF - 1 < n_chunks))(refill)

            wait_gather(b)
            add_comb(b, cc)
            start_write(b, cc)

        def round_body(rr, cc):
            for b in range(NBUF):
                process(rr * NBUF + b, b)
            return cc

        lax.fori_loop(0, n_rounds, round_body, 0)
        for t in range(n_tail):
            process(n_rounds * NBUF + t, t)

        for b in range(NBUF):
            wait_write(b)

    return k(tok_flat, seg_flat, word_embeddings, comb_flat)


def kernel(input_tokens, input_seg, word_embeddings, segment_embeddings):
    b, s = input_tokens.shape
    d = word_embeddings.shape[1]
    comb = _comb_table(segment_embeddings, s)
    tok_flat = input_tokens.reshape(-1).astype(jnp.int32)
    seg_flat = input_seg.reshape(-1).astype(jnp.int32)
    out = _sc_embed(tok_flat, seg_flat, word_embeddings, comb.reshape(-1), s)
    return out.reshape(b, s, d)


# speedup vs baseline: 1.5111x; 1.5111x over previous
import functools

import jax
import jax.numpy as jnp
from jax import lax
from jax.experimental import pallas as pl
from jax.experimental.pallas import tpu as pltpu
from jax.experimental.pallas import tpu_sc as plsc

NC = 2
NS = 16
LANES = 16
NBUF = 3


def _comb_table(segment_embeddings, seq_len):
    n_seg, d = segment_embeddings.shape

    def body(seg_ref, out_ref):
        s_idx = lax.broadcasted_iota(jnp.int32, (seq_len, d), 0)
        d_idx = lax.broadcasted_iota(jnp.int32, (seq_len, d), 1)
        i2 = ((d_idx // 2) * 2).astype(jnp.float32)
        div = jnp.exp(-jnp.log(10000.0) * i2 / d)
        ang = s_idx.astype(jnp.float32) * div
        pe = jnp.where(d_idx % 2 == 0, jnp.sin(ang), jnp.cos(ang))
        for t in range(n_seg):
            out_ref[pl.ds(t * seq_len, seq_len), :] = pe + seg_ref[t:t + 1, :]

    return pl.pallas_call(
        body,
        out_shape=jax.ShapeDtypeStruct((n_seg * seq_len, d), jnp.float32),
    )(segment_embeddings)


def _sc_embed(tok_flat, seg_flat, word_embeddings, comb_flat, seq_len):
    n = tok_flat.shape[0]
    d = word_embeddings.shape[1]
    nw = NC * NS
    rows_per_w = n // nw
    ch = 160
    n_chunks = rows_per_w // ch
    n_rounds, n_tail = divmod(n_chunks, NBUF)
    comb_elems = comb_flat.shape[0]
    assert rows_per_w % ch == 0 and ch % LANES == 0 and n_chunks >= NBUF

    mesh = plsc.VectorSubcoreMesh(
        core_axis_name="c", subcore_axis_name="s",
        num_cores=NC, num_subcores=NS)

    @functools.partial(
        pl.kernel,
        out_type=jax.ShapeDtypeStruct((n, d), jnp.float32),
        mesh=mesh,
        scratch_types=[
            pltpu.VMEM((rows_per_w,), jnp.int32),
            pltpu.VMEM((rows_per_w + LANES,), jnp.int32),
            pltpu.VMEM((comb_elems,), jnp.float32),
            [pltpu.VMEM((ch, d), jnp.float32) for _ in range(NBUF)],
            [pltpu.SemaphoreType.DMA for _ in range(NBUF)],
            [pltpu.SemaphoreType.DMA for _ in range(NBUF)],
        ],
    )
    def k(tok_hbm, seg_hbm, wtab_hbm, comb_hbm, out_hbm,
          tokb, segb, combl, bufs, gsems, wsems):
        wid = lax.axis_index("s") * NC + lax.axis_index("c")
        base = wid * rows_per_w

        pltpu.sync_copy(tok_hbm.at[pl.ds(base, rows_per_w)], tokb)

        def start_gather(b, c):
            pltpu.async_copy(
                wtab_hbm.at[tokb.at[pl.ds(c * ch, ch)]], bufs[b], gsems[b])

        def wait_gather(b):
            pltpu.make_async_copy(
                wtab_hbm.at[pl.ds(0, ch)], bufs[b], gsems[b]).wait()

        def start_write(b, c):
            pltpu.async_copy(
                bufs[b], out_hbm.at[pl.ds(base + c * ch, ch)], wsems[b])

        def wait_write(b):
            pltpu.make_async_copy(
                bufs[b], out_hbm.at[pl.ds(base, ch)], wsems[b]).wait()

        def add_comb(b, c):
            buf = bufs[b]

            @plsc.parallel_loop(0, ch, unroll=4)
            def _(row):
                flat = c * ch + row
                seg_s = segb[pl.ds(flat, LANES)][0]
                off = (seg_s * seq_len + lax.rem(flat, seq_len)) * d
                for j in range(d // LANES):
                    cv = combl[pl.ds(off + j * LANES, LANES)]
                    plsc.addupdate(buf.at[row, pl.ds(j * LANES, LANES)], cv)

        for b in range(NBUF):
            start_gather(b, b)
        pltpu.sync_copy(comb_hbm, combl)
        pltpu.sync_copy(seg_hbm.at[pl.ds(base, rows_per_w)],
                        segb.at[pl.ds(0, rows_per_w)])

        def process(cc, b):
            prev = (b - 1) % NBUF

            def refill():
                wait_write(prev)
                start_gather(prev, cc + NBUF - 1)

            if isinstance(cc, int):
                if 1 <= cc and cc + NBUF - 1 < n_chunks:
                    refill()
            else:
                pl.when(jnp.logical_and(cc >= 1, cc + NBUF - 1 < n_chunks))(refill)

            wait_gather(b)
            add_comb(b, cc)
            start_write(b, cc)

        def round_body(rr, cc):
            for b in range(NBUF):
                process(rr * NBUF + b, b)
            return cc

        lax.fori_loop(0, n_rounds, round_body, 0)
        for t in range(n_tail):
            process(n_rounds * NBUF + t, t)

        for b in range(NBUF):
            wait_write(b)

    return k(tok_flat, seg_flat, word_embeddings, comb_flat)


def kernel(input_tokens, input_seg, word_embeddings, segment_embeddings):
    b, s = input_tokens.shape
    d = word_embeddings.shape[1]
    comb = _comb_table(segment_embeddings, s)
    tok_flat = input_tokens.reshape(-1).astype(jnp.int32)
    seg_flat = input_seg.reshape(-1).astype(jnp.int32)
    out = _sc_embed(tok_flat, seg_flat, word_embeddings, comb.reshape(-1), s)
    return out.reshape(b, s, d)
